# baseline (speedup 1.0000x reference)
import jax
import jax.numpy as jnp
from jax import lax
from jax.experimental import pallas as pl
from jax.experimental.pallas import tpu as pltpu
from jax.experimental.pallas import tpu_sc as plsc

NC = 2
NS = 16
NW = NC * NS
LANES = 16
CHUNK = 80
BLK = 400


def _make_edge_agg(n_nodes, d, n_edges, idx_mult):
    e_per_w = n_edges // NW
    eb = 1280
    nblk = e_per_w // eb
    n_sub = eb // CHUNK
    stripe = (-(-(n_nodes // NS) // 8)) * 8
    last = n_nodes - (NS - 1) * stripe
    use_typ = idx_mult != 0

    mesh = plsc.VectorSubcoreMesh(
        core_axis_name="c", subcore_axis_name="s",
        num_cores=NC, num_subcores=NS)

    scratch = [
        pltpu.VMEM((n_sub, CHUNK), jnp.int32),
        pltpu.VMEM((n_sub, CHUNK), jnp.int32) if use_typ else None,
        pltpu.VMEM((n_sub, CHUNK), jnp.float32),
        pltpu.VMEM((n_sub, CHUNK), jnp.int32),
        pltpu.VMEM((CHUNK, d), jnp.float32),
        pltpu.VMEM((CHUNK, d), jnp.float32),
        pltpu.VMEM((CHUNK, d), jnp.float32),
        pltpu.VMEM((CHUNK, d), jnp.float32),
        pltpu.VMEM_SHARED((n_nodes, d), jnp.float32),
        pltpu.SemaphoreType.DMA,
        pltpu.SemaphoreType.DMA,
        pltpu.SemaphoreType.DMA,
        pltpu.SemaphoreType.DMA,
        pltpu.SemaphoreType.DMA,
    ]
    scratch = [s for s in scratch if s is not None]

    def body(*refs):
        if use_typ:
            (table_h, ei_h, typ_h, norm_h, zeros_h, out_h,
             src_v, typ_v, norm_v, dst_v, rows_a, rows_b, rows_c, rows_d,
             acc, gsem_a, gsem_b, gsem_c, gsem_d, ssem) = refs
        else:
            (table_h, ei_h, norm_h, zeros_h, out_h,
             src_v, norm_v, dst_v, rows_a, rows_b, rows_c, rows_d,
             acc, gsem_a, gsem_b, gsem_c, gsem_d, ssem) = refs
        bufs = (rows_a, rows_b, rows_c, rows_d)
        gsems = (gsem_a, gsem_b, gsem_c, gsem_d)
        cid = lax.axis_index("c")
        sid = lax.axis_index("s")
        wid = cid * NS + sid

        @pl.when(sid < NS - 1)
        def _():
            pltpu.sync_copy(zeros_h.at[pl.ds(sid * stripe, stripe)],
                            acc.at[pl.ds(sid * stripe, stripe)])

        @pl.when(sid == NS - 1)
        def _():
            pltpu.sync_copy(zeros_h.at[pl.ds((NS - 1) * stripe, last)],
                            acc.at[pl.ds((NS - 1) * stripe, last)])

        plsc.subcore_barrier()

        def blk_body(b, c0):
            pltpu.sync_copy(ei_h.at[0, wid, b], src_v)
            pltpu.sync_copy(ei_h.at[1, wid, b], dst_v)
            if use_typ:
                pltpu.sync_copy(typ_h.at[wid, b], typ_v)
            pltpu.sync_copy(norm_h.at[wid, b], norm_v)

            if use_typ:
                def idx_body(ii, c):
                    for q in range(CHUNK // LANES):
                        sl = pl.ds(q * LANES, LANES)
                        src_v[ii, sl] = typ_v[ii, sl] * idx_mult + src_v[ii, sl]
                    return c
                lax.fori_loop(0, n_sub, idx_body, 0)

            pltpu.async_copy(table_h.at[src_v.at[0]], bufs[0], gsems[0])
            pltpu.async_copy(table_h.at[src_v.at[1]], bufs[1], gsems[1])
            pltpu.async_copy(table_h.at[src_v.at[2]], bufs[2], gsems[2])

            def step(j, k):
                cur, pre = bufs[k], bufs[(k + 3) % 4]
                @pl.when(j > 0)
                def _():
                    pltpu.make_async_copy(
                        pre, acc.at[dst_v.at[0]], ssem).wait()

                @pl.when(j + 3 < n_sub)
                def _():
                    pltpu.async_copy(table_h.at[src_v.at[j + 3]], pre,
                                     gsems[(k + 3) % 4])

                pltpu.make_async_copy(
                    table_h.at[src_v.at[0]], cur, gsems[k]).wait()

                def scale_body(i16, cc):
                    norms = norm_v[j, pl.ds(i16 * LANES, LANES)]
                    for l in range(LANES):
                        nv = jnp.full((LANES,), norms[l], dtype=jnp.float32)
                        row = i16 * LANES + l
                        for col in range(d // LANES):
                            sl = pl.ds(col * LANES, LANES)
                            cur[row, sl] = cur[row, sl] * nv
                    return cc
                lax.fori_loop(0, CHUNK // LANES, scale_body, 0)

                pltpu.async_copy(cur, acc.at[dst_v.at[j]], ssem, add=True)

            def quad_body(jj, c):
                for u in range(4):
                    step(4 * jj + u, u)
                return c
            lax.fori_loop(0, n_sub // 4, quad_body, 0)
            for t in range(n_sub - n_sub % 4, n_sub):
                step(t, t % 4)
            pltpu.make_async_copy(
                bufs[(n_sub - 1) % 4], acc.at[dst_v.at[0]], ssem).wait()
            return c0
        lax.fori_loop(0, nblk, blk_body, 0)

        plsc.subcore_barrier()

        @pl.when(sid < NS - 1)
        def _():
            pltpu.sync_copy(acc.at[pl.ds(sid * stripe, stripe)],
                            out_h.at[pl.ds(cid * n_nodes + sid * stripe, stripe)])

        @pl.when(sid == NS - 1)
        def _():
            pltpu.sync_copy(
                acc.at[pl.ds((NS - 1) * stripe, last)],
                out_h.at[pl.ds(cid * n_nodes + (NS - 1) * stripe, last)])

    return pl.kernel(
        body,
        out_type=jax.ShapeDtypeStruct((NC * n_nodes, d), jnp.float32),
        mesh=mesh,
        scratch_types=scratch,
    )


def _rel_transform(x, w_rel):
    n, d = x.shape
    r, _, h = w_rel.shape
    tblk = 2000
    nb = n // tblk

    def body(x_ref, w_ref, o_ref):
        o_ref[0] = jnp.dot(x_ref[...], w_ref[0],
                           preferred_element_type=jnp.float32)

    return pl.pallas_call(
        body,
        grid=(nb, r),
        in_specs=[pl.BlockSpec((tblk, d), lambda ni, ri: (ni, 0)),
                  pl.BlockSpec((1, d, h), lambda ni, ri: (ri, 0, 0))],
        out_specs=pl.BlockSpec((1, tblk, h), lambda ni, ri: (ri, ni, 0)),
        out_shape=jax.ShapeDtypeStruct((r, n, h), jnp.float32),
    )(x, w_rel)


def _dense_mid(acc, x, w_root, b1, w2):
    n, d = x.shape
    h = w_root.shape[1]
    nb = n // BLK

    def body(a0, a1, x_ref, wr, b_ref, w2_ref, h1_ref, g_ref):
        h1 = (a0[...] + a1[...]
              + jnp.dot(x_ref[...], wr[...], preferred_element_type=jnp.float32)
              + b_ref[...])
        h1_ref[...] = h1
        g_ref[...] = jnp.dot(h1, w2_ref[...], preferred_element_type=jnp.float32)

    return pl.pallas_call(
        body,
        grid=(nb,),
        in_specs=[pl.BlockSpec((BLK, h), lambda i: (i, 0)),
                  pl.BlockSpec((BLK, h), lambda i: (i + nb, 0)),
                  pl.BlockSpec((BLK, d), lambda i: (i, 0)),
                  pl.BlockSpec((d, h), lambda i: (0, 0)),
                  pl.BlockSpec((1, h), lambda i: (0, 0)),
                  pl.BlockSpec((h, h), lambda i: (0, 0))],
        out_specs=[pl.BlockSpec((BLK, h), lambda i: (i, 0)),
                   pl.BlockSpec((BLK, h), lambda i: (i, 0))],
        out_shape=[jax.ShapeDtypeStruct((n, h), jnp.float32),
                   jax.ShapeDtypeStruct((n, h), jnp.float32)],
    )(acc, acc, x, w_root, b1.reshape(1, h), w2)


def _dense_out(acc, h1, x, w2_root, b2):
    n, d = x.shape
    h = w2_root.shape[1]
    nb = n // BLK

    def body(a0, a1, h1_ref, x_ref, w_ref, b_ref, o_ref):
        h2 = (a0[...] + a1[...]
              + jnp.dot(h1_ref[...], w_ref[...], preferred_element_type=jnp.float32)
              + b_ref[...])
        o_ref[...] = jnp.concatenate([x_ref[...], jnp.maximum(h2, 0.0)], axis=1)

    return pl.pallas_call(
        body,
        grid=(nb,),
        in_specs=[pl.BlockSpec((BLK, h), lambda i: (i, 0)),
                  pl.BlockSpec((BLK, h), lambda i: (i + nb, 0)),
                  pl.BlockSpec((BLK, h), lambda i: (i, 0)),
                  pl.BlockSpec((BLK, d), lambda i: (i, 0)),
                  pl.BlockSpec((h, h), lambda i: (0, 0)),
                  pl.BlockSpec((1, h), lambda i: (0, 0))],
        out_specs=pl.BlockSpec((BLK, d + h), lambda i: (i, 0)),
        out_shape=jax.ShapeDtypeStruct((n, d + h), jnp.float32),
    )(acc, acc, h1, x, w2_root, b2.reshape(1, h))


def kernel(x, edge_index, edge_norm, edge_type, W_rel, W_root, b1, W2, W2_root, b2):
    n, d = x.shape
    h = W_root.shape[1]
    r = W_rel.shape[0]
    e = edge_index.shape[1]

    e_per_w = -(-e // (NW * 1280)) * 1280
    ep = NW * e_per_w
    nblk = e_per_w // 1280
    n_sub = 1280 // CHUNK
    ei = jnp.pad(edge_index.astype(jnp.int32), ((0, 0), (0, ep - e)))
    ei = ei.reshape(2, NW, nblk, n_sub, CHUNK)
    typ4 = jnp.pad(edge_type.astype(jnp.int32),
                   (0, ep - e)).reshape(NW, nblk, n_sub, CHUNK)
    norm4 = jnp.pad(edge_norm.astype(jnp.float32),
                    (0, ep - e)).reshape(NW, nblk, n_sub, CHUNK)
    zeros = jnp.zeros((n, h), jnp.float32)

    xw = _rel_transform(x, W_rel).reshape(r * n, h)
    agg1 = _make_edge_agg(n, h, ep, idx_mult=n)(xw, ei, typ4, norm4, zeros)
    h1, g = _dense_mid(agg1, x, W_root, b1, W2)
    agg2 = _make_edge_agg(n, h, ep, idx_mult=0)(g, ei, norm4, zeros)
    return _dense_out(agg2, h1, x, W2_root, b2)


# speedup vs baseline: 2.7329x; 2.7329x over previous
import jax
import jax.numpy as jnp
from jax import lax
from jax.experimental import pallas as pl
from jax.experimental.pallas import tpu as pltpu
from jax.experimental.pallas import tpu_sc as plsc

NC = 2
NS = 16
NW = NC * NS
LANES = 16
CHUNK = 80
BLK = 400


def _make_edge_agg(n_nodes, d, n_edges, idx_mult):
    e_per_w = n_edges // NW
    eb = 2000
    nblk = e_per_w // eb
    n_sub = eb // CHUNK
    stripe = (-(-(n_nodes // NS) // 8)) * 8
    last = n_nodes - (NS - 1) * stripe
    use_typ = idx_mult != 0

    mesh = plsc.VectorSubcoreMesh(
        core_axis_name="c", subcore_axis_name="s",
        num_cores=NC, num_subcores=NS)

    scratch = [
        pltpu.VMEM((n_sub, CHUNK), jnp.int32),
        pltpu.VMEM((n_sub, CHUNK), jnp.int32) if use_typ else None,
        pltpu.VMEM((n_sub, CHUNK), jnp.float32),
        pltpu.VMEM((n_sub, CHUNK), jnp.int32),
        pltpu.VMEM((CHUNK, d), jnp.float32),
        pltpu.VMEM((CHUNK, d), jnp.float32),
        pltpu.VMEM((CHUNK, d), jnp.float32),
        pltpu.VMEM_SHARED((n_nodes, d), jnp.float32),
        pltpu.SemaphoreType.DMA,
        pltpu.SemaphoreType.DMA,
        pltpu.SemaphoreType.DMA,
        pltpu.SemaphoreType.DMA,
    ]
    scratch = [s for s in scratch if s is not None]

    def body(*refs):
        if use_typ:
            (table_h, ei_h, typ_h, norm_h, zeros_h, out_h,
             src_v, typ_v, norm_v, dst_v, rows_a, rows_b, rows_c, acc,
             gsem_a, gsem_b, gsem_c, ssem) = refs
        else:
            (table_h, ei_h, norm_h, zeros_h, out_h,
             src_v, norm_v, dst_v, rows_a, rows_b, rows_c, acc,
             gsem_a, gsem_b, gsem_c, ssem) = refs
        bufs = (rows_a, rows_b, rows_c)
        gsems = (gsem_a, gsem_b, gsem_c)
        cid = lax.axis_index("c")
        sid = lax.axis_index("s")
        wid = cid * NS + sid

        @pl.when(sid < NS - 1)
        def _():
            pltpu.sync_copy(zeros_h.at[pl.ds(sid * stripe, stripe)],
                            acc.at[pl.ds(sid * stripe, stripe)])

        @pl.when(sid == NS - 1)
        def _():
            pltpu.sync_copy(zeros_h.at[pl.ds((NS - 1) * stripe, last)],
                            acc.at[pl.ds((NS - 1) * stripe, last)])

        plsc.subcore_barrier()

        def blk_body(b, c0):
            pltpu.sync_copy(ei_h.at[0, wid, b], src_v)
            pltpu.sync_copy(ei_h.at[1, wid, b], dst_v)
            if use_typ:
                pltpu.sync_copy(typ_h.at[wid, b], typ_v)
            pltpu.sync_copy(norm_h.at[wid, b], norm_v)

            if use_typ:
                def idx_body(ii, c):
                    for q in range(CHUNK // LANES):
                        sl = pl.ds(q * LANES, LANES)
                        src_v[ii, sl] = typ_v[ii, sl] * idx_mult + src_v[ii, sl]
                    return c
                lax.fori_loop(0, n_sub, idx_body, 0)

            pltpu.async_copy(table_h.at[src_v.at[0]], bufs[0], gsems[0])
            pltpu.async_copy(table_h.at[src_v.at[1]], bufs[1], gsems[1])

            def step(j, k):
                cur, pre = bufs[k], bufs[(k + 2) % 3]
                @pl.when(j > 0)
                def _():
                    pltpu.make_async_copy(
                        pre, acc.at[dst_v.at[0]], ssem).wait()

                @pl.when(j + 2 < n_sub)
                def _():
                    pltpu.async_copy(table_h.at[src_v.at[j + 2]], pre,
                                     gsems[(k + 2) % 3])

                pltpu.make_async_copy(
                    table_h.at[src_v.at[0]], cur, gsems[k]).wait()

                def scale_body(i16, cc):
                    norms = norm_v[j, pl.ds(i16 * LANES, LANES)]
                    for l in range(LANES):
                        nv = jnp.full((LANES,), norms[l], dtype=jnp.float32)
                        row = i16 * LANES + l
                        for col in range(d // LANES):
                            sl = pl.ds(col * LANES, LANES)
                            cur[row, sl] = cur[row, sl] * nv
                    return cc
                lax.fori_loop(0, CHUNK // LANES, scale_body, 0)

                pltpu.async_copy(cur, acc.at[dst_v.at[j]], ssem, add=True)

            def tri_body(jj, c):
                step(3 * jj, 0)
                step(3 * jj + 1, 1)
                step(3 * jj + 2, 2)
                return c
            lax.fori_loop(0, n_sub // 3, tri_body, 0)
            for t in range(n_sub - n_sub % 3, n_sub):
                step(t, t % 3)
            pltpu.make_async_copy(
                bufs[(n_sub - 1) % 3], acc.at[dst_v.at[0]], ssem).wait()
            return c0
        lax.fori_loop(0, nblk, blk_body, 0)

        plsc.subcore_barrier()

        @pl.when(sid < NS - 1)
        def _():
            pltpu.sync_copy(acc.at[pl.ds(sid * stripe, stripe)],
                            out_h.at[pl.ds(cid * n_nodes + sid * stripe, stripe)])

        @pl.when(sid == NS - 1)
        def _():
            pltpu.sync_copy(
                acc.at[pl.ds((NS - 1) * stripe, last)],
                out_h.at[pl.ds(cid * n_nodes + (NS - 1) * stripe, last)])

    return pl.kernel(
        body,
        out_type=jax.ShapeDtypeStruct((NC * n_nodes, d), jnp.float32),
        mesh=mesh,
        scratch_types=scratch,
    )


def _rel_transform(x, w_rel):
    n, d = x.shape
    r, _, h = w_rel.shape
    tblk = 2000
    nb = n // tblk

    def body(x_ref, w_ref, o_ref):
        o_ref[0] = jnp.dot(x_ref[...], w_ref[0],
                           preferred_element_type=jnp.float32)

    return pl.pallas_call(
        body,
        grid=(nb, r),
        in_specs=[pl.BlockSpec((tblk, d), lambda ni, ri: (ni, 0)),
                  pl.BlockSpec((1, d, h), lambda ni, ri: (ri, 0, 0))],
        out_specs=pl.BlockSpec((1, tblk, h), lambda ni, ri: (ri, ni, 0)),
        out_shape=jax.ShapeDtypeStruct((r, n, h), jnp.float32),
    )(x, w_rel)


def _dense_mid(acc, x, w_root, b1, w2):
    n, d = x.shape
    h = w_root.shape[1]
    nb = n // BLK

    def body(a0, a1, x_ref, wr, b_ref, w2_ref, h1_ref, g_ref):
        h1 = (a0[...] + a1[...]
              + jnp.dot(x_ref[...], wr[...], preferred_element_type=jnp.float32)
              + b_ref[...])
        h1_ref[...] = h1
        g_ref[...] = jnp.dot(h1, w2_ref[...], preferred_element_type=jnp.float32)

    return pl.pallas_call(
        body,
        grid=(nb,),
        in_specs=[pl.BlockSpec((BLK, h), lambda i: (i, 0)),
                  pl.BlockSpec((BLK, h), lambda i: (i + nb, 0)),
                  pl.BlockSpec((BLK, d), lambda i: (i, 0)),
                  pl.BlockSpec((d, h), lambda i: (0, 0)),
                  pl.BlockSpec((1, h), lambda i: (0, 0)),
                  pl.BlockSpec((h, h), lambda i: (0, 0))],
        out_specs=[pl.BlockSpec((BLK, h), lambda i: (i, 0)),
                   pl.BlockSpec((BLK, h), lambda i: (i, 0))],
        out_shape=[jax.ShapeDtypeStruct((n, h), jnp.float32),
                   jax.ShapeDtypeStruct((n, h), jnp.float32)],
    )(acc, acc, x, w_root, b1.reshape(1, h), w2)


def _dense_out(acc, h1, x, w2_root, b2):
    n, d = x.shape
    h = w2_root.shape[1]
    nb = n // BLK

    def body(a0, a1, h1_ref, x_ref, w_ref, b_ref, o_ref):
        h2 = (a0[...] + a1[...]
              + jnp.dot(h1_ref[...], w_ref[...], preferred_element_type=jnp.float32)
              + b_ref[...])
        o_ref[...] = jnp.concatenate([x_ref[...], jnp.maximum(h2, 0.0)], axis=1)

    return pl.pallas_call(
        body,
        grid=(nb,),
        in_specs=[pl.BlockSpec((BLK, h), lambda i: (i, 0)),
                  pl.BlockSpec((BLK, h), lambda i: (i + nb, 0)),
                  pl.BlockSpec((BLK, h), lambda i: (i, 0)),
                  pl.BlockSpec((BLK, d), lambda i: (i, 0)),
                  pl.BlockSpec((h, h), lambda i: (0, 0)),
                  pl.BlockSpec((1, h), lambda i: (0, 0))],
        out_specs=pl.BlockSpec((BLK, d + h), lambda i: (i, 0)),
        out_shape=jax.ShapeDtypeStruct((n, d + h), jnp.float32),
    )(acc, acc, h1, x, w2_root, b2.reshape(1, h))


def kernel(x, edge_index, edge_norm, edge_type, W_rel, W_root, b1, W2, W2_root, b2):
    n, d = x.shape
    h = W_root.shape[1]
    r = W_rel.shape[0]
    e = edge_index.shape[1]

    e_per_w = e // NW
    nblk = e_per_w // 2000
    n_sub = 2000 // CHUNK
    ei = edge_index.astype(jnp.int32).reshape(2, NW, nblk, n_sub, CHUNK)
    typ4 = edge_type.astype(jnp.int32).reshape(NW, nblk, n_sub, CHUNK)
    norm4 = edge_norm.astype(jnp.float32).reshape(NW, nblk, n_sub, CHUNK)
    zeros = jnp.zeros((n, h), jnp.float32)

    xw = _rel_transform(x, W_rel).reshape(r * n, h)
    agg1 = _make_edge_agg(n, h, e, idx_mult=n)(xw, ei, typ4, norm4, zeros)
    h1, g = _dense_mid(agg1, x, W_root, b1, W2)
    agg2 = _make_edge_agg(n, h, e, idx_mult=0)(g, ei, norm4, zeros)
    return _dense_out(agg2, h1, x, W2_root, b2)
